# baseline (speedup 1.0000x reference)
import jax
import jax.numpy as jnp
from jax.experimental import pallas as pl
from jax.experimental.pallas import tpu as pltpu

N_TOK = 32768
FEAT_DIM = 768
TEXT_DIM = 768
NUM_EXPERTS = 8
HID = 128
SUB = 1024
DEPTH = 4
NSTEPS = N_TOK // SUB
OUT_ROWS = SUB * NUM_EXPERTS // 128


def _router_body(temp_ref, f_hbm, t_hbm, fmu_ref, frho_ref, tmu_ref, trho_ref,
                 cmu_ref, crho_ref, ef_ref, et_ref, ec_ref,
                 probs_ref, logits_ref,
                 fbuf, tbuf, fw_s, tw_s, cw_s, t8_s, mask_s, sems):
    i = pl.program_id(0)

    def _fcopy(blk, slot):
        return pltpu.make_async_copy(
            f_hbm.at[pl.ds(blk * SUB, SUB), :], fbuf.at[slot], sems.at[0, slot])

    def _tcopy(blk, slot):
        return pltpu.make_async_copy(
            t_hbm.at[pl.ds(blk * SUB, SUB), :], tbuf.at[slot], sems.at[1, slot])

    @pl.when(i == 0)
    def _():
        for d in range(DEPTH):
            _fcopy(d, d).start()
            _tcopy(d, d).start()
        fw_s[...] = fmu_ref[...] + jnp.log(1.0 + jnp.exp(frho_ref[...])) * ef_ref[...]
        tw_s[...] = tmu_ref[...] + jnp.log(1.0 + jnp.exp(trho_ref[...])) * et_ref[...]
        cw_s[...] = cmu_ref[...] + jnp.log(1.0 + jnp.exp(crho_ref[...])) * ec_ref[...]
        lane8 = jax.lax.broadcasted_iota(jnp.int32, (NUM_EXPERTS, 128), 1)
        e8 = jax.lax.broadcasted_iota(jnp.int32, (NUM_EXPERTS, 128), 0)
        t8_s[...] = (jax.lax.rem(lane8, NUM_EXPERTS) == e8).astype(jnp.float32)
        lane = jax.lax.broadcasted_iota(jnp.int32, (SUB, 128), 1)
        row = jax.lax.broadcasted_iota(jnp.int32, (SUB, 128), 0)
        mask_s[...] = (jax.lax.rem(row, 128 // NUM_EXPERTS)
                       == lane // NUM_EXPERTS).astype(jnp.float32)

    slot = jax.lax.rem(i, DEPTH)
    _fcopy(i, slot).wait()
    _tcopy(i, slot).wait()

    cw = cw_s[...]
    fp = jnp.dot(fbuf[slot], fw_s[...], preferred_element_type=jnp.float32)
    tp = jnp.dot(tbuf[slot], tw_s[...], preferred_element_type=jnp.float32)
    logits = (jnp.dot(fp, cw[:HID, :], preferred_element_type=jnp.float32)
              + jnp.dot(tp, cw[HID:, :], preferred_element_type=jnp.float32))
    inv_t = 1.0 / jnp.maximum(temp_ref[0, 0], 0.1)
    logits = logits * inv_t
    def _pack(x):
        tiled = jnp.dot(x, t8_s[...], preferred_element_type=jnp.float32)
        b = (tiled * mask_s[...]).reshape(OUT_ROWS, 128 // NUM_EXPERTS, 128)
        return jnp.sum(b, axis=1)

    logits_ref[...] = _pack(logits)
    m = jnp.max(logits, axis=1, keepdims=True)
    e = jnp.exp(logits - m)
    probs = e / jnp.sum(e, axis=1, keepdims=True)
    probs_ref[...] = _pack(probs)

    nxt = i + DEPTH

    @pl.when(nxt < NSTEPS)
    def _():
        _fcopy(nxt, slot).start()
        _tcopy(nxt, slot).start()


def kernel(feature, text_embedding, feature_mu, feature_rho, text_mu, text_rho,
           combined_mu, combined_rho, temperature, epsilon_f, epsilon_t, epsilon_c):
    temp2d = temperature.reshape(1, 1)
    full = lambda shape: pl.BlockSpec(shape, lambda i: (0, 0))
    hbm = pl.BlockSpec(memory_space=pltpu.MemorySpace.HBM)
    probs, logits = pl.pallas_call(
        _router_body,
        grid=(NSTEPS,),
        in_specs=[
            full((1, 1)),
            hbm,
            hbm,
            full((FEAT_DIM, HID)),
            full((FEAT_DIM, HID)),
            full((TEXT_DIM, HID)),
            full((TEXT_DIM, HID)),
            full((2 * HID, NUM_EXPERTS)),
            full((2 * HID, NUM_EXPERTS)),
            full((FEAT_DIM, HID)),
            full((TEXT_DIM, HID)),
            full((2 * HID, NUM_EXPERTS)),
        ],
        out_specs=[
            pl.BlockSpec((OUT_ROWS, 128), lambda i: (i, 0)),
            pl.BlockSpec((OUT_ROWS, 128), lambda i: (i, 0)),
        ],
        out_shape=[
            jax.ShapeDtypeStruct((N_TOK * NUM_EXPERTS // 128, 128), jnp.float32),
            jax.ShapeDtypeStruct((N_TOK * NUM_EXPERTS // 128, 128), jnp.float32),
        ],
        scratch_shapes=[
            pltpu.VMEM((DEPTH, SUB, FEAT_DIM), jnp.float32),
            pltpu.VMEM((DEPTH, SUB, TEXT_DIM), jnp.float32),
            pltpu.VMEM((FEAT_DIM, HID), jnp.float32),
            pltpu.VMEM((TEXT_DIM, HID), jnp.float32),
            pltpu.VMEM((2 * HID, NUM_EXPERTS), jnp.float32),
            pltpu.VMEM((NUM_EXPERTS, 128), jnp.float32),
            pltpu.VMEM((SUB, 128), jnp.float32),
            pltpu.SemaphoreType.DMA((2, DEPTH)),
        ],
        compiler_params=pltpu.CompilerParams(
            dimension_semantics=("arbitrary",),
            vmem_limit_bytes=120 * 1024 * 1024,
        ),
    )(temp2d, feature, text_embedding, feature_mu, feature_rho, text_mu,
      text_rho, combined_mu, combined_rho, epsilon_f, epsilon_t, epsilon_c)
    return (probs.reshape(N_TOK, NUM_EXPERTS), logits.reshape(N_TOK, NUM_EXPERTS))


# speedup vs baseline: 1.1779x; 1.1779x over previous
import jax
import jax.numpy as jnp
from jax.experimental import pallas as pl
from jax.experimental.pallas import tpu as pltpu

N_TOK = 32768
FEAT_DIM = 768
TEXT_DIM = 768
NUM_EXPERTS = 8
HID = 128
SUB = 1024
DEPTH = 4
NSTEPS = N_TOK // SUB


def _router_body(temp_ref, f_hbm, t_hbm, fmu_ref, frho_ref, tmu_ref, trho_ref,
                 cmu_ref, crho_ref, ef_ref, et_ref, ec_ref,
                 probs_hbm, logits_hbm,
                 fbuf, tbuf, fw_s, tw_s, cw_s, pstage, lstage, sems, osems):
    i = pl.program_id(0)

    def _fcopy(blk, slot):
        return pltpu.make_async_copy(
            f_hbm.at[pl.ds(blk * SUB, SUB), :], fbuf.at[slot], sems.at[0, slot])

    def _tcopy(blk, slot):
        return pltpu.make_async_copy(
            t_hbm.at[pl.ds(blk * SUB, SUB), :], tbuf.at[slot], sems.at[1, slot])

    def _pcopy(blk, ss):
        return pltpu.make_async_copy(
            pstage.at[ss], probs_hbm.at[pl.ds(blk * SUB, SUB), :], osems.at[0, ss])

    def _lcopy(blk, ss):
        return pltpu.make_async_copy(
            lstage.at[ss], logits_hbm.at[pl.ds(blk * SUB, SUB), :], osems.at[1, ss])

    @pl.when(i == 0)
    def _():
        for d in range(DEPTH):
            _fcopy(d, d).start()
            _tcopy(d, d).start()
        fw_s[...] = fmu_ref[...] + jnp.log(1.0 + jnp.exp(frho_ref[...])) * ef_ref[...]
        tw_s[...] = tmu_ref[...] + jnp.log(1.0 + jnp.exp(trho_ref[...])) * et_ref[...]
        cw_s[...] = cmu_ref[...] + jnp.log(1.0 + jnp.exp(crho_ref[...])) * ec_ref[...]

    slot = jax.lax.rem(i, DEPTH)
    ss = jax.lax.rem(i, 2)
    _fcopy(i, slot).wait()
    _tcopy(i, slot).wait()

    cw = cw_s[...]
    fp = jnp.dot(fbuf[slot], fw_s[...], preferred_element_type=jnp.float32)
    tp = jnp.dot(tbuf[slot], tw_s[...], preferred_element_type=jnp.float32)
    logits = (jnp.dot(fp, cw[:HID, :], preferred_element_type=jnp.float32)
              + jnp.dot(tp, cw[HID:, :], preferred_element_type=jnp.float32))
    inv_t = 1.0 / jnp.maximum(temp_ref[0, 0], 0.1)
    logits = logits * inv_t
    m = jnp.max(logits, axis=1, keepdims=True)
    e = jnp.exp(logits - m)
    probs = e / jnp.sum(e, axis=1, keepdims=True)

    @pl.when(i >= 2)
    def _():
        _pcopy(i - 2, ss).wait()
        _lcopy(i - 2, ss).wait()

    lstage[ss] = logits
    pstage[ss] = probs
    _pcopy(i, ss).start()
    _lcopy(i, ss).start()

    nxt = i + DEPTH

    @pl.when(nxt < NSTEPS)
    def _():
        _fcopy(nxt, slot).start()
        _tcopy(nxt, slot).start()

    @pl.when(i == NSTEPS - 1)
    def _():
        _pcopy(i - 1, 1 - ss).wait()
        _lcopy(i - 1, 1 - ss).wait()
        _pcopy(i, ss).wait()
        _lcopy(i, ss).wait()


def kernel(feature, text_embedding, feature_mu, feature_rho, text_mu, text_rho,
           combined_mu, combined_rho, temperature, epsilon_f, epsilon_t, epsilon_c):
    temp2d = temperature.reshape(1, 1)
    full = lambda shape: pl.BlockSpec(shape, lambda i: (0, 0))
    hbm = pl.BlockSpec(memory_space=pltpu.MemorySpace.HBM)
    probs, logits = pl.pallas_call(
        _router_body,
        grid=(NSTEPS,),
        in_specs=[
            full((1, 1)),
            hbm,
            hbm,
            full((FEAT_DIM, HID)),
            full((FEAT_DIM, HID)),
            full((TEXT_DIM, HID)),
            full((TEXT_DIM, HID)),
            full((2 * HID, NUM_EXPERTS)),
            full((2 * HID, NUM_EXPERTS)),
            full((FEAT_DIM, HID)),
            full((TEXT_DIM, HID)),
            full((2 * HID, NUM_EXPERTS)),
        ],
        out_specs=[hbm, hbm],
        out_shape=[
            jax.ShapeDtypeStruct((N_TOK, NUM_EXPERTS), jnp.float32),
            jax.ShapeDtypeStruct((N_TOK, NUM_EXPERTS), jnp.float32),
        ],
        scratch_shapes=[
            pltpu.VMEM((DEPTH, SUB, FEAT_DIM), jnp.float32),
            pltpu.VMEM((DEPTH, SUB, TEXT_DIM), jnp.float32),
            pltpu.VMEM((FEAT_DIM, HID), jnp.float32),
            pltpu.VMEM((TEXT_DIM, HID), jnp.float32),
            pltpu.VMEM((2 * HID, NUM_EXPERTS), jnp.float32),
            pltpu.VMEM((2, SUB, NUM_EXPERTS), jnp.float32),
            pltpu.VMEM((2, SUB, NUM_EXPERTS), jnp.float32),
            pltpu.SemaphoreType.DMA((2, DEPTH)),
            pltpu.SemaphoreType.DMA((2, 2)),
        ],
        compiler_params=pltpu.CompilerParams(
            dimension_semantics=("arbitrary",),
            vmem_limit_bytes=120 * 1024 * 1024,
        ),
    )(temp2d, feature, text_embedding, feature_mu, feature_rho, text_mu,
      text_rho, combined_mu, combined_rho, epsilon_f, epsilon_t, epsilon_c)
    return (probs, logits)


# speedup vs baseline: 1.1932x; 1.0130x over previous
import jax
import jax.numpy as jnp
from jax.experimental import pallas as pl
from jax.experimental.pallas import tpu as pltpu

N_TOK = 32768
FEAT_DIM = 768
TEXT_DIM = 768
NUM_EXPERTS = 8
HID = 128
SUB = 1024
DEPTH = 4
NSTEPS = N_TOK // SUB


def _router_body(temp_ref, f_hbm, t_hbm, fmu_ref, frho_ref, tmu_ref, trho_ref,
                 cmu_ref, crho_ref, ef_ref, et_ref, ec_ref,
                 probs_ref, logits_ref,
                 fbuf, tbuf, a_s, b_s, negm_s, sems):
    i = pl.program_id(0)

    def _fcopy(blk, slot):
        return pltpu.make_async_copy(
            f_hbm.at[pl.ds(blk * SUB, SUB), :], fbuf.at[slot], sems.at[0, slot])

    def _tcopy(blk, slot):
        return pltpu.make_async_copy(
            t_hbm.at[pl.ds(blk * SUB, SUB), :], tbuf.at[slot], sems.at[1, slot])

    @pl.when(i == 0)
    def _():
        for d in range(DEPTH):
            _fcopy(d, d).start()
            _tcopy(d, d).start()
        fw = fmu_ref[...] + jnp.log(1.0 + jnp.exp(frho_ref[...])) * ef_ref[...]
        tw = tmu_ref[...] + jnp.log(1.0 + jnp.exp(trho_ref[...])) * et_ref[...]
        cw = cmu_ref[...] + jnp.log(1.0 + jnp.exp(crho_ref[...])) * ec_ref[...]
        lane = jax.lax.broadcasted_iota(jnp.int32, (HID, 128), 1)
        cwf = jnp.where(lane < NUM_EXPERTS,
                        jnp.dot(cw[:HID, :],
                                (jax.lax.broadcasted_iota(jnp.int32, (NUM_EXPERTS, 128), 0)
                                 == jax.lax.broadcasted_iota(jnp.int32, (NUM_EXPERTS, 128), 1)
                                 ).astype(jnp.float32),
                                preferred_element_type=jnp.float32), 0.0)
        cwt = jnp.where(lane < NUM_EXPERTS,
                        jnp.dot(cw[HID:, :],
                                (jax.lax.broadcasted_iota(jnp.int32, (NUM_EXPERTS, 128), 0)
                                 == jax.lax.broadcasted_iota(jnp.int32, (NUM_EXPERTS, 128), 1)
                                 ).astype(jnp.float32),
                                preferred_element_type=jnp.float32), 0.0)
        a_s[...] = jnp.dot(fw, cwf, preferred_element_type=jnp.float32)
        b_s[...] = jnp.dot(tw, cwt, preferred_element_type=jnp.float32)
        lane2 = jax.lax.broadcasted_iota(jnp.int32, (SUB, 128), 1)
        negm_s[...] = jnp.where(lane2 < NUM_EXPERTS, 0.0, -1e30)

    slot = jax.lax.rem(i, DEPTH)
    _fcopy(i, slot).wait()
    _tcopy(i, slot).wait()

    logits = (jnp.dot(fbuf[slot], a_s[...], preferred_element_type=jnp.float32)
              + jnp.dot(tbuf[slot], b_s[...], preferred_element_type=jnp.float32))
    inv_t = 1.0 / jnp.maximum(temp_ref[0, 0], 0.1)
    logits = logits * inv_t
    logits_ref[...] = logits
    masked = logits + negm_s[...]
    m = jnp.max(masked, axis=1, keepdims=True)
    e = jnp.exp(masked - m)
    probs_ref[...] = e / jnp.sum(e, axis=1, keepdims=True)

    nxt = i + DEPTH

    @pl.when(nxt < NSTEPS)
    def _():
        _fcopy(nxt, slot).start()
        _tcopy(nxt, slot).start()


def kernel(feature, text_embedding, feature_mu, feature_rho, text_mu, text_rho,
           combined_mu, combined_rho, temperature, epsilon_f, epsilon_t, epsilon_c):
    temp2d = temperature.reshape(1, 1)
    full = lambda shape: pl.BlockSpec(shape, lambda i: (0, 0))
    hbm = pl.BlockSpec(memory_space=pltpu.MemorySpace.HBM)
    probs128, logits128 = pl.pallas_call(
        _router_body,
        grid=(NSTEPS,),
        in_specs=[
            full((1, 1)),
            hbm,
            hbm,
            full((FEAT_DIM, HID)),
            full((FEAT_DIM, HID)),
            full((TEXT_DIM, HID)),
            full((TEXT_DIM, HID)),
            full((2 * HID, NUM_EXPERTS)),
            full((2 * HID, NUM_EXPERTS)),
            full((FEAT_DIM, HID)),
            full((TEXT_DIM, HID)),
            full((2 * HID, NUM_EXPERTS)),
        ],
        out_specs=[
            pl.BlockSpec((SUB, 128), lambda i: (i, 0)),
            pl.BlockSpec((SUB, 128), lambda i: (i, 0)),
        ],
        out_shape=[
            jax.ShapeDtypeStruct((N_TOK, 128), jnp.float32),
            jax.ShapeDtypeStruct((N_TOK, 128), jnp.float32),
        ],
        scratch_shapes=[
            pltpu.VMEM((DEPTH, SUB, FEAT_DIM), jnp.float32),
            pltpu.VMEM((DEPTH, SUB, TEXT_DIM), jnp.float32),
            pltpu.VMEM((FEAT_DIM, 128), jnp.float32),
            pltpu.VMEM((TEXT_DIM, 128), jnp.float32),
            pltpu.VMEM((SUB, 128), jnp.float32),
            pltpu.SemaphoreType.DMA((2, DEPTH)),
        ],
        compiler_params=pltpu.CompilerParams(
            dimension_semantics=("arbitrary",),
            vmem_limit_bytes=120 * 1024 * 1024,
        ),
    )(temp2d, feature, text_embedding, feature_mu, feature_rho, text_mu,
      text_rho, combined_mu, combined_rho, epsilon_f, epsilon_t, epsilon_c)
    return (probs128[:, :NUM_EXPERTS], logits128[:, :NUM_EXPERTS])
